# baseline (speedup 1.0000x reference)
import functools

import jax
import jax.numpy as jnp
from jax import lax
from jax.experimental import pallas as pl
from jax.experimental.pallas import tpu as pltpu
from jax.experimental.pallas import tpu_sc as plsc

B = 4096
DEG = 50
DEGP = 56
D = 64
NC = 2
NS = 16
NW = NC * NS
IDX_W = 128
ROWS_PER_W = (B * DEGP) // NW // IDX_W
EPW = ROWS_PER_W * IDX_W
NBUF = 3
GROUP = 4
GROWS = GROUP * IDX_W
NG = ROWS_PER_W // GROUP
UPW = B // NW

BB = 256
NBLK = BB * DEGP


def _sc_gather(item_emb, idx2, user_emb, nodes2):
    mesh = plsc.VectorSubcoreMesh(
        core_axis_name="c", subcore_axis_name="s",
        num_cores=NC, num_subcores=NS)

    @functools.partial(
        pl.kernel,
        out_type=(
            jax.ShapeDtypeStruct((B * DEGP, D), jnp.float32),
            jax.ShapeDtypeStruct((B, D), jnp.float32),
        ),
        mesh=mesh,
        compiler_params=pltpu.CompilerParams(use_tc_tiling_on_sc=False),
        scratch_types=(
            pltpu.VMEM((ROWS_PER_W, IDX_W), jnp.int32),
            pltpu.VMEM((NBUF, GROWS, D), jnp.float32),
            pltpu.VMEM((UPW,), jnp.int32),
            pltpu.VMEM((UPW, D), jnp.float32),
            pltpu.SemaphoreType.DMA((NBUF,)),
            pltpu.SemaphoreType.DMA((NBUF,)),
            pltpu.SemaphoreType.DMA,
        ),
    )
    def k(item_hbm, idx_hbm, user_hbm, nodes_hbm, g_hbm, u_hbm,
          idx_v, bufs, uidx_v, urows_v, gsems, osems, usem):
        wid = lax.axis_index("s") * NC + lax.axis_index("c")
        pltpu.sync_copy(idx_hbm.at[pl.ds(wid * ROWS_PER_W, ROWS_PER_W)], idx_v)
        pltpu.sync_copy(nodes_hbm.at[wid], uidx_v)
        ucp = pltpu.async_copy(user_hbm.at[uidx_v], urows_v, usem)
        obase = wid * EPW

        def fire_g(g, b):
            for j in range(GROUP):
                pltpu.async_copy(
                    item_hbm.at[idx_v.at[g * GROUP + j]],
                    bufs.at[b, pl.ds(j * IDX_W, IDX_W)], gsems.at[b])

        def drain_g(b):
            pltpu.make_async_copy(
                item_hbm.at[pl.ds(0, GROWS)], bufs.at[b], gsems.at[b]).wait()

        def fire_o(g, b):
            pltpu.async_copy(
                bufs.at[b], g_hbm.at[pl.ds(obase + g * GROWS, GROWS)],
                osems.at[b])

        def drain_o(b):
            pltpu.make_async_copy(
                bufs.at[b], g_hbm.at[pl.ds(obase, GROWS)], osems.at[b]).wait()

        fire_g(0, 0)
        fire_g(1, 1)
        drain_g(0)
        fire_o(0, 0)
        fire_g(2, 2)

        def body(g, carry):
            b = lax.rem(g, NBUF)
            bp = lax.rem(g + 2, NBUF)
            drain_g(b)
            fire_o(g, b)
            drain_o(bp)
            fire_g(g + 2, bp)
            return carry

        lax.fori_loop(1, NG - 2, body, 0)
        drain_g((NG - 2) % NBUF)
        fire_o(NG - 2, (NG - 2) % NBUF)
        drain_o((NG - 3) % NBUF)
        drain_g((NG - 1) % NBUF)
        fire_o(NG - 1, (NG - 1) % NBUF)
        drain_o((NG - 2) % NBUF)
        drain_o((NG - 1) % NBUF)
        ucp.wait()
        pltpu.sync_copy(urows_v, u_hbm.at[pl.ds(wid * UPW, UPW)])

    return k(item_emb, idx2, user_emb, nodes2)


def _tc_body(g_ref, rid_ref, u_ref, w1_ref, w1b_ref, w2_ref, w2b_ref,
             wa1_ref, wa1b_ref, wa2_ref, wa2b_ref, wa3_ref, wa3b_ref,
             remb_ref, out_ref):
    f32 = jnp.float32
    g = g_ref[...]
    ids = rid_ref[...]
    w1a = w1_ref[0:D, :]
    r1 = jnp.dot(remb_ref[...], w1_ref[D:2 * D, :],
                 preferred_element_type=f32)
    oh = jnp.where(lax.broadcasted_iota(jnp.int32, (NBLK, 8), 1) == ids,
                   f32(1.0), f32(0.0))
    rc = jnp.dot(oh, r1, preferred_element_type=f32,
                 precision=lax.Precision.HIGHEST)

    t = jnp.maximum(jnp.dot(g, w1a, preferred_element_type=f32)
                    + rc + w1b_ref[...], 0.0)
    uv_r = jnp.maximum(jnp.dot(t, w2_ref[...], preferred_element_type=f32)
                       + w2b_ref[...], 0.0)

    self_c = jnp.dot(u_ref[...], wa1_ref[D:2 * D, :],
                     preferred_element_type=f32)
    h1 = jnp.dot(uv_r, wa1_ref[0:D, :], preferred_element_type=f32)
    h = jnp.maximum(h1.reshape(BB, DEGP, D) + self_c[:, None, :]
                    + wa1b_ref[...][None, :, :], 0.0)
    h2 = jnp.maximum(jnp.dot(h.reshape(NBLK, D), wa2_ref[...],
                             preferred_element_type=f32)
                     + wa2b_ref[...], 0.0)
    logits = (jnp.sum(h2 * wa3_ref[...], axis=-1, keepdims=True)
              + wa3b_ref[...])

    l3 = logits.reshape(BB, DEGP, 1)
    pos = lax.broadcasted_iota(jnp.int32, (BB, DEGP, 1), 1)
    l3 = jnp.where(pos < DEG, l3, f32(-1e30))
    m = jnp.max(l3, axis=1, keepdims=True)
    e = jnp.exp(l3 - m)
    s = jnp.sum(e, axis=1)
    num = jnp.sum(uv_r.reshape(BB, DEGP, D) * e, axis=1)
    out_ref[...] = num / s


def _tc_mlp(g, rid, u, w1_w, w1_b, w2_w, w2_b, wa1_w, wa1_b, wa2_w, wa2_b,
            wa3r, wa3_b, remb):
    grid = (B // BB,)
    full = lambda shape: pl.BlockSpec(shape, lambda i: (0, 0))
    return pl.pallas_call(
        _tc_body,
        grid=grid,
        in_specs=[
            pl.BlockSpec((NBLK, D), lambda i: (i, 0)),
            pl.BlockSpec((NBLK, 1), lambda i: (i, 0)),
            pl.BlockSpec((BB, D), lambda i: (i, 0)),
            full((2 * D, D)), full((1, D)),
            full((D, D)), full((1, D)),
            full((2 * D, D)), full((1, D)),
            full((D, D)), full((1, D)),
            full((1, D)), full((1, 1)),
            full((8, D)),
        ],
        out_specs=pl.BlockSpec((BB, D), lambda i: (i, 0)),
        out_shape=jax.ShapeDtypeStruct((B, D), jnp.float32),
    )(g, rid, u, w1_w, w1_b, w2_w, w2_b, wa1_w, wa1_b, wa2_w, wa2_b,
      wa3r, wa3_b, remb)


def kernel(nodes, uv_adjacency, ratings, user_emb, item_emb, rating_emb,
           w1_w, w1_b, w2_w, w2_b, wa1_w, wa1_b, wa2_w, wa2_b, wa3_w, wa3_b):
    npad = DEGP - DEG
    pad_idx = (lax.broadcasted_iota(jnp.int32, (B, npad), 0) * npad
               + lax.broadcasted_iota(jnp.int32, (B, npad), 1))
    adj_p = jnp.concatenate([uv_adjacency.astype(jnp.int32), pad_idx], axis=1)
    idx2 = adj_p.reshape(B * DEGP // IDX_W, IDX_W)
    rat_p = jnp.pad(ratings.astype(jnp.int32), ((0, 0), (0, DEGP - DEG)))
    rid = rat_p.reshape(B * DEGP, 1)
    nodes2 = nodes.astype(jnp.int32).reshape(NW, UPW)

    g, u = _sc_gather(item_emb, idx2, user_emb, nodes2)

    remb = jnp.pad(rating_emb, ((0, 3), (0, 0)))
    return _tc_mlp(
        g, rid, u,
        w1_w, w1_b.reshape(1, D),
        w2_w, w2_b.reshape(1, D),
        wa1_w, wa1_b.reshape(1, D),
        wa2_w, wa2_b.reshape(1, D),
        wa3_w.reshape(1, D), wa3_b.reshape(1, 1),
        remb)


# speedup vs baseline: 1.3729x; 1.3729x over previous
import functools

import jax
import jax.numpy as jnp
from jax import lax
from jax.experimental import pallas as pl
from jax.experimental.pallas import tpu as pltpu
from jax.experimental.pallas import tpu_sc as plsc

B = 4096
DEG = 50
DEGP = 56
D = 64
NC = 2
NS = 16
NW = NC * NS
IDX_W = 128
ROWS_PER_W = (B * DEGP) // NW // IDX_W
EPW = ROWS_PER_W * IDX_W
NBUF = 3
GROUP = 4
GROWS = GROUP * IDX_W
NG = ROWS_PER_W // GROUP
UPW = B // NW

BB = 256
NBLK = BB * DEGP


def _sc_gather(item_emb, idx2, user_emb, nodes2):
    mesh = plsc.VectorSubcoreMesh(
        core_axis_name="c", subcore_axis_name="s",
        num_cores=NC, num_subcores=NS)

    @functools.partial(
        pl.kernel,
        out_type=(
            jax.ShapeDtypeStruct((B * DEGP, D), jnp.float32),
            jax.ShapeDtypeStruct((B, D), jnp.float32),
        ),
        mesh=mesh,
        compiler_params=pltpu.CompilerParams(use_tc_tiling_on_sc=False),
        scratch_types=(
            pltpu.VMEM((ROWS_PER_W, IDX_W), jnp.int32),
            pltpu.VMEM((NBUF, GROWS, D), jnp.float32),
            pltpu.VMEM((UPW,), jnp.int32),
            pltpu.VMEM((UPW, D), jnp.float32),
            pltpu.SemaphoreType.DMA((NBUF,)),
            pltpu.SemaphoreType.DMA((NBUF,)),
            pltpu.SemaphoreType.DMA,
        ),
    )
    def k(item_hbm, idx_hbm, user_hbm, nodes_hbm, g_hbm, u_hbm,
          idx_v, bufs, uidx_v, urows_v, gsems, osems, usem):
        wid = lax.axis_index("s") * NC + lax.axis_index("c")
        pltpu.sync_copy(idx_hbm.at[pl.ds(wid * ROWS_PER_W, ROWS_PER_W)], idx_v)
        pltpu.sync_copy(nodes_hbm.at[wid], uidx_v)
        ucp = pltpu.async_copy(user_hbm.at[uidx_v], urows_v, usem)
        obase = wid * EPW

        def fire_g(g, b):
            for j in range(GROUP):
                pltpu.async_copy(
                    item_hbm.at[idx_v.at[g * GROUP + j]],
                    bufs.at[b, pl.ds(j * IDX_W, IDX_W)], gsems.at[b])

        def drain_g(b):
            pltpu.make_async_copy(
                item_hbm.at[pl.ds(0, GROWS)], bufs.at[b], gsems.at[b]).wait()

        def fire_o(g, b):
            pltpu.async_copy(
                bufs.at[b], g_hbm.at[pl.ds(obase + g * GROWS, GROWS)],
                osems.at[b])

        def drain_o(b):
            pltpu.make_async_copy(
                bufs.at[b], g_hbm.at[pl.ds(obase, GROWS)], osems.at[b]).wait()

        fire_g(0, 0)
        fire_g(1, 1)
        drain_g(0)
        fire_o(0, 0)
        fire_g(2, 2)

        def body(g, carry):
            b = lax.rem(g, NBUF)
            bp = lax.rem(g + 2, NBUF)
            drain_g(b)
            fire_o(g, b)
            drain_o(bp)
            fire_g(g + 2, bp)
            return carry

        lax.fori_loop(1, NG - 2, body, 0)
        drain_g((NG - 2) % NBUF)
        fire_o(NG - 2, (NG - 2) % NBUF)
        drain_o((NG - 3) % NBUF)
        drain_g((NG - 1) % NBUF)
        fire_o(NG - 1, (NG - 1) % NBUF)
        drain_o((NG - 2) % NBUF)
        drain_o((NG - 1) % NBUF)
        ucp.wait()
        pltpu.sync_copy(urows_v, u_hbm.at[pl.ds(wid * UPW, UPW)])

    return k(item_emb, idx2, user_emb, nodes2)


def _tc_body(g_ref, rid_ref, u_ref, w1_ref, w1b_ref, w2_ref, w2b_ref,
             wa1_ref, wa1b_ref, wa2_ref, wa2b_ref, wa3_ref, wa3b_ref,
             remb_ref, out_ref):
    f32 = jnp.float32
    g = g_ref[...]
    ids = rid_ref[...]
    w1a = w1_ref[0:D, :]
    r1 = jnp.dot(remb_ref[...], w1_ref[D:2 * D, :],
                 preferred_element_type=f32)
    b0 = lax.bitwise_and(ids, 1) == 1
    b1 = lax.bitwise_and(ids, 2) == 2
    b2 = ids >= 4
    a01 = jnp.where(b0, r1[1:2, :], r1[0:1, :])
    a23 = jnp.where(b0, r1[3:4, :], r1[2:3, :])
    a03 = jnp.where(b1, a23, a01)
    rc = jnp.where(b2, r1[4:5, :], a03)

    t = jnp.maximum(jnp.dot(g, w1a, preferred_element_type=f32)
                    + rc + w1b_ref[...], 0.0)
    uv_r = jnp.maximum(jnp.dot(t, w2_ref[...], preferred_element_type=f32)
                       + w2b_ref[...], 0.0)

    self_c = jnp.dot(u_ref[...], wa1_ref[D:2 * D, :],
                     preferred_element_type=f32)
    h1 = jnp.dot(uv_r, wa1_ref[0:D, :], preferred_element_type=f32)
    h = jnp.maximum(h1.reshape(BB, DEGP, D) + self_c[:, None, :]
                    + wa1b_ref[...][None, :, :], 0.0)
    h2 = jnp.maximum(jnp.dot(h.reshape(NBLK, D), wa2_ref[...],
                             preferred_element_type=f32)
                     + wa2b_ref[...], 0.0)
    logits = (jnp.sum(h2 * wa3_ref[...], axis=-1, keepdims=True)
              + wa3b_ref[...])

    l3 = logits.reshape(BB, DEGP, 1)
    pos = lax.broadcasted_iota(jnp.int32, (BB, DEGP, 1), 1)
    l3 = jnp.where(pos < DEG, l3, f32(-1e30))
    m = jnp.max(l3, axis=1, keepdims=True)
    e = jnp.exp(l3 - m)
    s = jnp.sum(e, axis=1)
    num = jnp.sum(uv_r.reshape(BB, DEGP, D) * e, axis=1)
    out_ref[...] = num / s


def _tc_mlp(g, rid, u, w1_w, w1_b, w2_w, w2_b, wa1_w, wa1_b, wa2_w, wa2_b,
            wa3r, wa3_b, remb):
    grid = (B // BB,)
    full = lambda shape: pl.BlockSpec(shape, lambda i: (0, 0))
    return pl.pallas_call(
        _tc_body,
        grid=grid,
        in_specs=[
            pl.BlockSpec((NBLK, D), lambda i: (i, 0)),
            pl.BlockSpec((NBLK, 1), lambda i: (i, 0)),
            pl.BlockSpec((BB, D), lambda i: (i, 0)),
            full((2 * D, D)), full((1, D)),
            full((D, D)), full((1, D)),
            full((2 * D, D)), full((1, D)),
            full((D, D)), full((1, D)),
            full((1, D)), full((1, 1)),
            full((8, D)),
        ],
        out_specs=pl.BlockSpec((BB, D), lambda i: (i, 0)),
        out_shape=jax.ShapeDtypeStruct((B, D), jnp.float32),
    )(g, rid, u, w1_w, w1_b, w2_w, w2_b, wa1_w, wa1_b, wa2_w, wa2_b,
      wa3r, wa3_b, remb)


def kernel(nodes, uv_adjacency, ratings, user_emb, item_emb, rating_emb,
           w1_w, w1_b, w2_w, w2_b, wa1_w, wa1_b, wa2_w, wa2_b, wa3_w, wa3_b):
    npad = DEGP - DEG
    pad_idx = (lax.broadcasted_iota(jnp.int32, (B, npad), 0) * npad
               + lax.broadcasted_iota(jnp.int32, (B, npad), 1))
    adj_p = jnp.concatenate([uv_adjacency.astype(jnp.int32), pad_idx], axis=1)
    idx2 = adj_p.reshape(B * DEGP // IDX_W, IDX_W)
    rat_p = jnp.pad(ratings.astype(jnp.int32), ((0, 0), (0, DEGP - DEG)))
    rid = rat_p.reshape(B * DEGP, 1)
    nodes2 = nodes.astype(jnp.int32).reshape(NW, UPW)

    g, u = _sc_gather(item_emb, idx2, user_emb, nodes2)

    remb = jnp.pad(rating_emb, ((0, 3), (0, 0)))
    return _tc_mlp(
        g, rid, u,
        w1_w, w1_b.reshape(1, D),
        w2_w, w2_b.reshape(1, D),
        wa1_w, wa1_b.reshape(1, D),
        wa2_w, wa2_b.reshape(1, D),
        wa3_w.reshape(1, D), wa3_b.reshape(1, 1),
        remb)


# speedup vs baseline: 2.0182x; 1.4700x over previous
import functools

import jax
import jax.numpy as jnp
from jax import lax
from jax.experimental import pallas as pl
from jax.experimental.pallas import tpu as pltpu
from jax.experimental.pallas import tpu_sc as plsc

B = 4096
DEG = 50
DEGP = 56
D = 64
NC = 2
NS = 16
NW = NC * NS
IDX_W = 128
ROWS_PER_W = (B * DEGP) // NW // IDX_W
EPW = ROWS_PER_W * IDX_W
NBUF = 3
GROUP = 4
GROWS = GROUP * IDX_W
NG = ROWS_PER_W // GROUP
UPW = B // NW

BB = 256
NBLK = BB * DEGP


def _sc_gather(item_emb, idx2, user_emb, nodes2):
    mesh = plsc.VectorSubcoreMesh(
        core_axis_name="c", subcore_axis_name="s",
        num_cores=NC, num_subcores=NS)

    @functools.partial(
        pl.kernel,
        out_type=(
            jax.ShapeDtypeStruct((B * DEGP, D), jnp.float32),
            jax.ShapeDtypeStruct((B, D), jnp.float32),
        ),
        mesh=mesh,
        compiler_params=pltpu.CompilerParams(use_tc_tiling_on_sc=False),
        scratch_types=(
            pltpu.VMEM((ROWS_PER_W, IDX_W), jnp.int32),
            pltpu.VMEM((NBUF, GROWS, D), jnp.float32),
            pltpu.VMEM((UPW,), jnp.int32),
            pltpu.VMEM((UPW, D), jnp.float32),
            pltpu.SemaphoreType.DMA((NBUF,)),
            pltpu.SemaphoreType.DMA((NBUF,)),
            pltpu.SemaphoreType.DMA,
        ),
    )
    def k(item_hbm, idx_hbm, user_hbm, nodes_hbm, g_hbm, u_hbm,
          idx_v, bufs, uidx_v, urows_v, gsems, osems, usem):
        wid = lax.axis_index("s") * NC + lax.axis_index("c")
        pltpu.sync_copy(idx_hbm.at[pl.ds(wid * ROWS_PER_W, ROWS_PER_W)], idx_v)
        pltpu.sync_copy(nodes_hbm.at[wid], uidx_v)
        ucp = pltpu.async_copy(user_hbm.at[uidx_v], urows_v, usem)
        obase = wid * EPW

        def fire_g(g, b):
            for j in range(GROUP):
                pltpu.async_copy(
                    item_hbm.at[idx_v.at[g * GROUP + j]],
                    bufs.at[b, pl.ds(j * IDX_W, IDX_W)], gsems.at[b])

        def drain_g(b):
            pltpu.make_async_copy(
                item_hbm.at[pl.ds(0, GROWS)], bufs.at[b], gsems.at[b]).wait()

        def fire_o(g, b):
            pltpu.async_copy(
                bufs.at[b], g_hbm.at[pl.ds(obase + g * GROWS, GROWS)],
                osems.at[b])

        def drain_o(b):
            pltpu.make_async_copy(
                bufs.at[b], g_hbm.at[pl.ds(obase, GROWS)], osems.at[b]).wait()

        fire_g(0, 0)
        fire_g(1, 1)
        drain_g(0)
        fire_o(0, 0)
        fire_g(2, 2)

        def body(g, carry):
            b = lax.rem(g, NBUF)
            bp = lax.rem(g + 2, NBUF)
            drain_g(b)
            fire_o(g, b)
            drain_o(bp)
            fire_g(g + 2, bp)
            return carry

        lax.fori_loop(1, NG - 2, body, 0)
        drain_g((NG - 2) % NBUF)
        fire_o(NG - 2, (NG - 2) % NBUF)
        drain_o((NG - 3) % NBUF)
        drain_g((NG - 1) % NBUF)
        fire_o(NG - 1, (NG - 1) % NBUF)
        drain_o((NG - 2) % NBUF)
        drain_o((NG - 1) % NBUF)
        ucp.wait()
        pltpu.sync_copy(urows_v, u_hbm.at[pl.ds(wid * UPW, UPW)])

    return k(item_emb, idx2, user_emb, nodes2)


DP = DEGP // 2
N2BLK = BB * DP


def _tc_body(gp_ref, ids_ref, u_ref, w1blk_ref, w2blk_ref, wa1blk_ref,
             wa2blk_ref, wa3blk_ref, wa1bh_ref, w1rh_ref, remb_ref,
             b1p_ref, b2p_ref, ba1p_ref, ba2p_ref, out_ref):
    f32 = jnp.float32
    D2 = 2 * D
    gp = gp_ref[...]
    ids = ids_ref[...]
    r1 = jnp.dot(remb_ref[...], w1rh_ref[...],
                 preferred_element_type=f32)
    r1p = jnp.concatenate([r1, r1], axis=1)

    lane = lax.broadcasted_iota(jnp.int32, (N2BLK, D2), 1)
    idsx = jnp.where(lane < D, ids[:, 0:1], ids[:, 1:2])
    b0 = lax.bitwise_and(idsx, 1) == 1
    b1 = lax.bitwise_and(idsx, 2) == 2
    b2 = idsx >= 4
    a01 = jnp.where(b0, r1p[1:2, :], r1p[0:1, :])
    a23 = jnp.where(b0, r1p[3:4, :], r1p[2:3, :])
    rc = jnp.where(b2, r1p[4:5, :], jnp.where(b1, a23, a01))

    t = jnp.maximum(jnp.dot(gp, w1blk_ref[...], preferred_element_type=f32)
                    + rc + b1p_ref[...], 0.0)
    uvr = jnp.maximum(jnp.dot(t, w2blk_ref[...], preferred_element_type=f32)
                      + b2p_ref[...], 0.0)

    selfc = jnp.dot(u_ref[...], wa1bh_ref[...],
                    preferred_element_type=f32)
    selfp = jnp.concatenate([selfc, selfc], axis=1)
    h1 = jnp.dot(uvr, wa1blk_ref[...], preferred_element_type=f32)
    h = jnp.maximum(h1.reshape(BB, DP, D2) + selfp[:, None, :]
                    + ba1p_ref[...][None, :, :], 0.0)
    h2 = jnp.maximum(
        jnp.dot(h.reshape(N2BLK, D2), wa2blk_ref[...],
                preferred_element_type=f32) + ba2p_ref[...], 0.0)
    lg = jnp.dot(h2, wa3blk_ref[...], preferred_element_type=f32)

    l3 = lg.reshape(BB, DP, 2)
    ki = lax.broadcasted_iota(jnp.int32, (BB, DP, 2), 1)
    hi = lax.broadcasted_iota(jnp.int32, (BB, DP, 2), 2)
    l3 = jnp.where(2 * ki + hi < DEG, l3, f32(-1e30))
    m = jnp.max(jnp.max(l3, axis=2, keepdims=True), axis=1, keepdims=True)
    e = jnp.exp(l3 - m)
    se = jnp.sum(e, axis=1)
    s = se[:, 0:1] + se[:, 1:2]
    lane3 = lax.broadcasted_iota(jnp.int32, (BB, DP, D2), 2)
    esel = jnp.where(lane3 < D, e[:, :, 0:1], e[:, :, 1:2])
    nump = jnp.sum(uvr.reshape(BB, DP, D2) * esel, axis=1)
    out_ref[...] = (nump[:, 0:D] + nump[:, D:D2]) / s


def _tc_mlp(gp, ids2, u, w1blk, w2blk, wa1blk, wa2blk, wa3blk, wa1bh, w1rh,
            remb, b1p, b2p, ba1p, ba2p):
    D2 = 2 * D
    full = lambda shape: pl.BlockSpec(shape, lambda i: (0, 0))
    return pl.pallas_call(
        _tc_body,
        grid=(B // BB,),
        in_specs=[
            pl.BlockSpec((N2BLK, D2), lambda i: (i, 0)),
            pl.BlockSpec((N2BLK, 2), lambda i: (i, 0)),
            pl.BlockSpec((BB, D), lambda i: (i, 0)),
            full((D2, D2)), full((D2, D2)), full((D2, D2)), full((D2, D2)),
            full((D2, 2)), full((D, D)), full((D, D)), full((8, D)),
            full((1, D2)), full((1, D2)), full((1, D2)), full((1, D2)),
        ],
        out_specs=pl.BlockSpec((BB, D), lambda i: (i, 0)),
        out_shape=jax.ShapeDtypeStruct((B, D), jnp.float32),
    )(gp, ids2, u, w1blk, w2blk, wa1blk, wa2blk, wa3blk, wa1bh, w1rh,
      remb, b1p, b2p, ba1p, ba2p)


def _blkdiag(w):
    z = jnp.zeros_like(w)
    return jnp.concatenate(
        [jnp.concatenate([w, z], axis=1), jnp.concatenate([z, w], axis=1)],
        axis=0)


def kernel(nodes, uv_adjacency, ratings, user_emb, item_emb, rating_emb,
           w1_w, w1_b, w2_w, w2_b, wa1_w, wa1_b, wa2_w, wa2_b, wa3_w, wa3_b):
    npad = DEGP - DEG
    pad_idx = (lax.broadcasted_iota(jnp.int32, (B, npad), 0) * npad
               + lax.broadcasted_iota(jnp.int32, (B, npad), 1))
    adj_p = jnp.concatenate([uv_adjacency.astype(jnp.int32), pad_idx], axis=1)
    idx2 = adj_p.reshape(B * DEGP // IDX_W, IDX_W)
    nodes2 = nodes.astype(jnp.int32).reshape(NW, UPW)

    g, u = _sc_gather(item_emb, idx2, user_emb, nodes2)
    gp = g.reshape(B * DEGP // 2, 2 * D)

    rat_p = jnp.pad(ratings.astype(jnp.int32), ((0, 0), (0, npad)))
    ids2 = rat_p.reshape(B * DEGP // 2, 2)

    remb = jnp.pad(rating_emb, ((0, 3), (0, 0)))
    pair = lambda v: jnp.concatenate([v, v], axis=0).reshape(1, 2 * D)
    return _tc_mlp(
        gp, ids2, u,
        _blkdiag(w1_w[:D]), _blkdiag(w2_w), _blkdiag(wa1_w[:D]),
        _blkdiag(wa2_w),
        _blkdiag(wa3_w),
        wa1_w[D:], w1_w[D:], remb,
        pair(w1_b), pair(w2_b), pair(wa1_b), pair(wa2_b))
